# Initial kernel scaffold; baseline (speedup 1.0000x reference)
import jax
import jax.numpy as jnp
from jax.experimental import pallas as pl


def kernel(edge_index, values, B):
    raise NotImplementedError("write your pallas kernel here")


# speedup vs baseline: 2.4469x; 2.4469x over previous
import jax
import jax.numpy as jnp
from jax import lax
from jax.experimental import pallas as pl
from jax.experimental.pallas import tpu as pltpu
from jax.experimental.pallas import tpu_sc as plsc

N = 10000
NP = 10240
E = 320000
D = 128
NC = 2
NS = 16
L = 16
EPT = E // NS
K = 80
NB = EPT // K
RPT = NP // NS
ZR = 128


def _body(cols_h, rows_h, vals_h, b_h, out_h,
          cidx, ridx, vbuf, gbuf, zbuf, acc, sem):
    c = lax.axis_index("c")
    s = lax.axis_index("s")

    @pl.when(c == 0)
    def _work():
        ebase = s * EPT

        zero = jnp.zeros((L,), jnp.float32)

        def zrow(i, carry):
            for q in range(D // L):
                zbuf[i, pl.ds(q * L, L)] = zero
            return carry

        lax.fori_loop(0, ZR, zrow, 0)
        rbase = s * RPT
        for z in range(RPT // ZR):
            pltpu.sync_copy(zbuf, acc.at[pl.ds(rbase + z * ZR, ZR)])
        plsc.subcore_barrier()

        def batch(j, carry):
            eoff = ebase + j * K
            esl = pl.ds(eoff, K)
            pltpu.sync_copy(cols_h.at[esl], cidx)
            pltpu.sync_copy(rows_h.at[esl], ridx)
            pltpu.sync_copy(vals_h.at[esl], vbuf)
            pltpu.async_copy(b_h.at[cidx], gbuf, sem).wait()
            for t in range(K // L):
                vvec = vbuf[pl.ds(t * L, L)]
                for u in range(L):
                    i = t * L + u
                    v = vvec[u]
                    for q in range(D // L):
                        sl = pl.ds(q * L, L)
                        gbuf[i, sl] = gbuf[i, sl] * v
            pltpu.sync_copy(gbuf, acc.at[ridx], add=True)
            return carry

        lax.fori_loop(0, NB, batch, 0)
        plsc.subcore_barrier()

        pltpu.sync_copy(acc.at[pl.ds(rbase, RPT)],
                        out_h.at[pl.ds(rbase, RPT)])


_spmm = pl.kernel(
    _body,
    out_type=jax.ShapeDtypeStruct((NP, D), jnp.float32),
    mesh=plsc.VectorSubcoreMesh(
        core_axis_name="c", subcore_axis_name="s",
        num_cores=NC, num_subcores=NS),
    scratch_types=[
        pltpu.VMEM((K,), jnp.int32),
        pltpu.VMEM((K,), jnp.int32),
        pltpu.VMEM((K,), jnp.float32),
        pltpu.VMEM((K, D), jnp.float32),
        pltpu.VMEM((ZR, D), jnp.float32),
        pltpu.VMEM_SHARED((NP, D), jnp.float32),
        pltpu.SemaphoreType.DMA,
    ],
)


def kernel(edge_index, values, B):
    rows = edge_index[0]
    cols = edge_index[1]
    b_padded = jnp.pad(B, ((0, NP - N), (0, 0)))
    out_padded = _spmm(cols, rows, values, b_padded)
    return out_padded[:N]


# speedup vs baseline: 4.4373x; 1.8135x over previous
import jax
import jax.numpy as jnp
from jax import lax
from jax.experimental import pallas as pl
from jax.experimental.pallas import tpu as pltpu
from jax.experimental.pallas import tpu_sc as plsc

N = 10000
NP = 10240
E = 320000
D = 128
NC = 2
NS = 16
L = 16
EPT = E // (NC * NS)
K = 80
NB = EPT // K
RPT = NP // NS
ZR = 128


def _body(cols_h, rows_h, vals_h, b_h, out_h,
          cidx, ridx, vbuf, gbuf, zbuf, acc, sem):
    c = lax.axis_index("c")
    s = lax.axis_index("s")
    ebase = (c * NS + s) * EPT

    zero = jnp.zeros((L,), jnp.float32)

    def zrow(i, carry):
        for q in range(D // L):
            zbuf[i, pl.ds(q * L, L)] = zero
        return carry

    lax.fori_loop(0, ZR, zrow, 0)
    rbase = s * RPT
    for z in range(RPT // ZR):
        pltpu.sync_copy(zbuf, acc.at[pl.ds(rbase + z * ZR, ZR)])
    plsc.subcore_barrier()

    def batch(j, carry):
        eoff = ebase + j * K
        esl = pl.ds(eoff, K)
        pltpu.sync_copy(cols_h.at[esl], cidx)
        pltpu.sync_copy(rows_h.at[esl], ridx)
        pltpu.sync_copy(vals_h.at[esl], vbuf)
        pltpu.async_copy(b_h.at[cidx], gbuf, sem).wait()
        for t in range(K // L):
            vvec = vbuf[pl.ds(t * L, L)]
            for u in range(L):
                i = t * L + u
                v = vvec[u]
                for q in range(D // L):
                    sl = pl.ds(q * L, L)
                    gbuf[i, sl] = gbuf[i, sl] * v
        pltpu.sync_copy(gbuf, acc.at[ridx], add=True)
        return carry

    lax.fori_loop(0, NB, batch, 0)
    plsc.subcore_barrier()

    pltpu.sync_copy(acc.at[pl.ds(rbase, RPT)],
                    out_h.at[pl.ds(c * NP + rbase, RPT)])


_spmm = pl.kernel(
    _body,
    out_type=jax.ShapeDtypeStruct((NC * NP, D), jnp.float32),
    mesh=plsc.VectorSubcoreMesh(
        core_axis_name="c", subcore_axis_name="s",
        num_cores=NC, num_subcores=NS),
    scratch_types=[
        pltpu.VMEM((K,), jnp.int32),
        pltpu.VMEM((K,), jnp.int32),
        pltpu.VMEM((K,), jnp.float32),
        pltpu.VMEM((K, D), jnp.float32),
        pltpu.VMEM((ZR, D), jnp.float32),
        pltpu.VMEM_SHARED((NP, D), jnp.float32),
        pltpu.SemaphoreType.DMA,
    ],
)


def _add_body(a_ref, b_ref, o_ref):
    o_ref[...] = a_ref[...] + b_ref[...]


_combine = pl.pallas_call(
    _add_body,
    out_shape=jax.ShapeDtypeStruct((NP, D), jnp.float32),
)


def kernel(edge_index, values, B):
    rows = edge_index[0]
    cols = edge_index[1]
    b_padded = jnp.pad(B, ((0, NP - N), (0, 0)))
    partials = _spmm(cols, rows, values, b_padded)
    out_padded = _combine(partials[:NP], partials[NP:])
    return out_padded[:N]


# speedup vs baseline: 7.9147x; 1.7837x over previous
import jax
import jax.numpy as jnp
from jax import lax
from jax.experimental import pallas as pl
from jax.experimental.pallas import tpu as pltpu
from jax.experimental.pallas import tpu_sc as plsc

N = 10000
NP = 10240
E = 320000
D = 128
NC = 2
NS = 16
L = 16
EPT = E // (NC * NS)
K = 80
NB = EPT // K
PAIRS = NB // 2
RPT = NP // NS
ZR = 32


def _body(cols_h, rows_h, vals_h, b_h, out_h,
          cidx0, cidx1, ridx0, ridx1, rsh0, rsh1, vbuf0, vbuf1,
          gbuf0, gbuf1, sbuf0, sbuf1, zbuf, acc,
          semg0, semg1, sems0, sems1, semi0, semi1, zsem):
    cidx = [cidx0, cidx1]
    ridx = [ridx0, ridx1]
    rsh = [rsh0, rsh1]
    vbuf = [vbuf0, vbuf1]
    gbuf = [gbuf0, gbuf1]
    sbuf = [sbuf0, sbuf1]
    semg = [semg0, semg1]
    sems = [sems0, sems1]
    semi = [semi0, semi1]

    c = lax.axis_index("c")
    s = lax.axis_index("s")
    ebase = (c * NS + s) * EPT
    rbase = s * RPT

    def stage(j, b):
        esl = pl.ds(ebase + j * K, K)
        pltpu.async_copy(cols_h.at[esl], cidx[b], semi[b])
        pltpu.async_copy(rows_h.at[esl], ridx[b], semi[b])
        pltpu.async_copy(vals_h.at[esl], vbuf[b], semi[b])

    def wait_stage(b):
        esl = pl.ds(0, K)
        pltpu.make_async_copy(cols_h.at[esl], cidx[b], semi[b]).wait()
        pltpu.make_async_copy(rows_h.at[esl], ridx[b], semi[b]).wait()
        pltpu.make_async_copy(vals_h.at[esl], vbuf[b], semi[b]).wait()

    def fire_gather(b):
        pltpu.async_copy(b_h.at[cidx[b]], gbuf[b], semg[b])

    def wait_gather(b):
        pltpu.make_async_copy(b_h.at[cidx[b]], gbuf[b], semg[b]).wait()

    def scale(b):
        for t in range(K // L):
            vvec = vbuf[b][pl.ds(t * L, L)]
            for u in range(L):
                i = t * L + u
                v = vvec[u]
                for q in range(D // L):
                    sl = pl.ds(q * L, L)
                    sbuf[b][i, sl] = gbuf[b][i, sl] * v
        for t in range(K // L):
            sl = pl.ds(t * L, L)
            rsh[b][sl] = ridx[b][sl]

    def fire_scatter(b):
        pltpu.async_copy(sbuf[b], acc.at[rsh[b]], sems[b], add=True)

    def wait_scatter(b):
        pltpu.make_async_copy(sbuf[b], acc.at[rsh[b]], sems[b]).wait()

    zero = jnp.zeros((L,), jnp.float32)

    def zrow(i, carry):
        for q in range(D // L):
            zbuf[i, pl.ds(q * L, L)] = zero
        return carry

    lax.fori_loop(0, ZR, zrow, 0)
    for z in range(RPT // ZR):
        pltpu.async_copy(zbuf, acc.at[pl.ds(rbase + z * ZR, ZR)], zsem)
    for z in range(RPT // ZR):
        pltpu.make_async_copy(zbuf, acc.at[pl.ds(rbase + z * ZR, ZR)],
                              zsem).wait()
    plsc.subcore_barrier()

    stage(0, 0)
    wait_stage(0)
    fire_gather(0)
    stage(1, 1)

    def pair(j2, carry):
        for b in range(2):
            j = 2 * j2 + b
            wait_gather(b)

            @pl.when(j2 >= 1)
            def _():
                wait_scatter(b)

            scale(b)
            fire_scatter(b)
            wait_stage(1 - b)
            fire_gather(1 - b)
            if b == 0:
                stage(j + 2, 0)
            else:
                @pl.when(j2 < PAIRS - 1)
                def _():
                    stage(j + 2, 1)
        return carry

    lax.fori_loop(0, PAIRS, pair, 0)

    wait_gather(0)
    wait_scatter(0)
    scale(0)
    fire_scatter(0)
    wait_scatter(0)
    wait_scatter(1)
    plsc.subcore_barrier()

    pltpu.sync_copy(acc.at[pl.ds(rbase, RPT)],
                    out_h.at[pl.ds(c * NP + rbase, RPT)])


_spmm = pl.kernel(
    _body,
    out_type=jax.ShapeDtypeStruct((NC * NP, D), jnp.float32),
    mesh=plsc.VectorSubcoreMesh(
        core_axis_name="c", subcore_axis_name="s",
        num_cores=NC, num_subcores=NS),
    scratch_types=[
        pltpu.VMEM((K,), jnp.int32),
        pltpu.VMEM((K,), jnp.int32),
        pltpu.VMEM((K,), jnp.int32),
        pltpu.VMEM((K,), jnp.int32),
        pltpu.VMEM((K,), jnp.int32),
        pltpu.VMEM((K,), jnp.int32),
        pltpu.VMEM((K,), jnp.float32),
        pltpu.VMEM((K,), jnp.float32),
        pltpu.VMEM((K, D), jnp.float32),
        pltpu.VMEM((K, D), jnp.float32),
        pltpu.VMEM((K, D), jnp.float32),
        pltpu.VMEM((K, D), jnp.float32),
        pltpu.VMEM((ZR, D), jnp.float32),
        pltpu.VMEM_SHARED((NP, D), jnp.float32),
        pltpu.SemaphoreType.DMA,
        pltpu.SemaphoreType.DMA,
        pltpu.SemaphoreType.DMA,
        pltpu.SemaphoreType.DMA,
        pltpu.SemaphoreType.DMA,
        pltpu.SemaphoreType.DMA,
        pltpu.SemaphoreType.DMA,
    ],
)


def _add_body(a_ref, b_ref, o_ref):
    o_ref[...] = a_ref[...] + b_ref[...]


_combine = pl.pallas_call(
    _add_body,
    out_shape=jax.ShapeDtypeStruct((NP, D), jnp.float32),
)


def kernel(edge_index, values, B):
    rows = edge_index[0]
    cols = edge_index[1]
    b_padded = jnp.pad(B, ((0, NP - N), (0, 0)))
    partials = _spmm(cols, rows, values, b_padded)
    out_padded = _combine(partials[:NP], partials[NP:])
    return out_padded[:N]


# speedup vs baseline: 10.5216x; 1.3294x over previous
import jax
import jax.numpy as jnp
from jax import lax
from jax.experimental import pallas as pl
from jax.experimental.pallas import tpu as pltpu
from jax.experimental.pallas import tpu_sc as plsc

N = 10000
NP = 10240
E = 320000
D = 128
NC = 2
NS = 16
L = 16
EPT = E // (NC * NS)
K = 80
NB = EPT // K
PAIRS = NB // 2
RPT = NP // NS
ZR = 32


def _body(cols_h, rows_h, vals_h, b_h, out0_h, out1_h,
          cidx0, cidx1, ridx0, ridx1, rsh0, rsh1, vbuf0, vbuf1,
          gbuf0, gbuf1, sbuf0, sbuf1, zbuf, acc,
          semg0, semg1, sems0, sems1, semi0, semi1, zsem):
    cidx = [cidx0, cidx1]
    ridx = [ridx0, ridx1]
    rsh = [rsh0, rsh1]
    vbuf = [vbuf0, vbuf1]
    gbuf = [gbuf0, gbuf1]
    sbuf = [sbuf0, sbuf1]
    semg = [semg0, semg1]
    sems = [sems0, sems1]
    semi = [semi0, semi1]

    c = lax.axis_index("c")
    s = lax.axis_index("s")
    ebase = (c * NS + s) * EPT
    rbase = s * RPT

    def stage(j, b):
        esl = pl.ds(ebase + j * K, K)
        pltpu.async_copy(cols_h.at[esl], cidx[b], semi[b])
        pltpu.async_copy(rows_h.at[esl], ridx[b], semi[b])
        pltpu.async_copy(vals_h.at[esl], vbuf[b], semi[b])

    def wait_stage(b):
        esl = pl.ds(0, K)
        pltpu.make_async_copy(cols_h.at[esl], cidx[b], semi[b]).wait()
        pltpu.make_async_copy(rows_h.at[esl], ridx[b], semi[b]).wait()
        pltpu.make_async_copy(vals_h.at[esl], vbuf[b], semi[b]).wait()

    def fire_gather(b):
        pltpu.async_copy(b_h.at[cidx[b]], gbuf[b], semg[b])

    def wait_gather(b):
        pltpu.make_async_copy(b_h.at[cidx[b]], gbuf[b], semg[b]).wait()

    def snapshot(b):
        vvecs = []
        for t in range(K // L):
            sl = pl.ds(t * L, L)
            rsh[b][sl] = ridx[b][sl]
            vvecs.append(vbuf[b][sl])
        return vvecs

    def scale(b, vvecs):
        for t in range(K // L):
            for u in range(L):
                i = t * L + u
                v = vvecs[t][u]
                for q in range(D // L):
                    sl = pl.ds(q * L, L)
                    sbuf[b][i, sl] = gbuf[b][i, sl] * v

    def fire_scatter(b):
        pltpu.async_copy(sbuf[b], acc.at[rsh[b]], sems[b], add=True)

    def wait_scatter(b):
        pltpu.make_async_copy(sbuf[b], acc.at[rsh[b]], sems[b]).wait()

    zero = jnp.zeros((L,), jnp.float32)

    def zrow(i, carry):
        for q in range(D // L):
            zbuf[i, pl.ds(q * L, L)] = zero
        return carry

    lax.fori_loop(0, ZR, zrow, 0)
    for z in range(RPT // ZR):
        pltpu.async_copy(zbuf, acc.at[pl.ds(rbase + z * ZR, ZR)], zsem)
    for z in range(RPT // ZR):
        pltpu.make_async_copy(zbuf, acc.at[pl.ds(rbase + z * ZR, ZR)],
                              zsem).wait()
    plsc.subcore_barrier()

    stage(0, 0)
    wait_stage(0)
    fire_gather(0)
    stage(1, 1)

    def pair(j2, carry):
        for b in range(2):
            j = 2 * j2 + b
            wait_gather(b)
            wait_stage(1 - b)
            fire_gather(1 - b)

            @pl.when(j2 >= 1)
            def _():
                wait_scatter(b)

            vvecs = snapshot(b)
            if b == 0:
                stage(j + 2, 0)
            else:
                @pl.when(j2 < PAIRS - 1)
                def _():
                    stage(j + 2, 1)
            scale(b, vvecs)
            fire_scatter(b)
        return carry

    lax.fori_loop(0, PAIRS, pair, 0)

    wait_gather(0)
    wait_scatter(0)
    vvecs = snapshot(0)
    scale(0, vvecs)
    fire_scatter(0)
    wait_scatter(0)
    wait_scatter(1)
    plsc.subcore_barrier()

    osl = pl.ds(rbase, RPT)

    @pl.when(c == 0)
    def _():
        pltpu.sync_copy(acc.at[osl], out0_h.at[osl])

    @pl.when(c == 1)
    def _():
        pltpu.sync_copy(acc.at[osl], out1_h.at[osl])


_spmm = pl.kernel(
    _body,
    out_type=(jax.ShapeDtypeStruct((NP, D), jnp.float32),
              jax.ShapeDtypeStruct((NP, D), jnp.float32)),
    mesh=plsc.VectorSubcoreMesh(
        core_axis_name="c", subcore_axis_name="s",
        num_cores=NC, num_subcores=NS),
    scratch_types=[
        pltpu.VMEM((K,), jnp.int32),
        pltpu.VMEM((K,), jnp.int32),
        pltpu.VMEM((K,), jnp.int32),
        pltpu.VMEM((K,), jnp.int32),
        pltpu.VMEM((K,), jnp.int32),
        pltpu.VMEM((K,), jnp.int32),
        pltpu.VMEM((K,), jnp.float32),
        pltpu.VMEM((K,), jnp.float32),
        pltpu.VMEM((K, D), jnp.float32),
        pltpu.VMEM((K, D), jnp.float32),
        pltpu.VMEM((K, D), jnp.float32),
        pltpu.VMEM((K, D), jnp.float32),
        pltpu.VMEM((ZR, D), jnp.float32),
        pltpu.VMEM_SHARED((NP, D), jnp.float32),
        pltpu.SemaphoreType.DMA,
        pltpu.SemaphoreType.DMA,
        pltpu.SemaphoreType.DMA,
        pltpu.SemaphoreType.DMA,
        pltpu.SemaphoreType.DMA,
        pltpu.SemaphoreType.DMA,
        pltpu.SemaphoreType.DMA,
    ],
)


def _add_body(a_ref, b_ref, o_ref):
    sl = pl.ds(0, N)
    o_ref[...] = a_ref[sl, :] + b_ref[sl, :]


_combine = pl.pallas_call(
    _add_body,
    out_shape=jax.ShapeDtypeStruct((N, D), jnp.float32),
)


def kernel(edge_index, values, B):
    rows = edge_index[0]
    cols = edge_index[1]
    p0, p1 = _spmm(cols, rows, values, B)
    return _combine(p0, p1)
